# Initial kernel scaffold; baseline (speedup 1.0000x reference)
import jax
import jax.numpy as jnp
from jax.experimental import pallas as pl


def kernel(nodes, positions, senders, receivers, W_pre0, W_pre1, W_pre2, W_post0, W_post1, W_post2, W_sc):
    raise NotImplementedError("write your pallas kernel here")


# speedup vs baseline: 1.4981x; 1.4981x over previous
import functools

import jax
import jax.numpy as jnp
import numpy as np
from jax import lax
from jax.experimental import pallas as pl
from jax.experimental.pallas import tpu as pltpu
from jax.experimental.pallas import tpu_sc as plsc

N = 10000
E = 160000
D = 128
DEN = 16.0
INV = 1.0 / np.sqrt(D)

NC = 2
NS = 16
NWORK = NC * NS

EPW = E // NWORK
WA = 1000
EPT = E // NS
WB = 80
SUB = 80
NSUB = WB // SUB
NWIN = EPT // WB
ROWS_PT = N // NS
ZROWS = 25
CPS = 4

_S3 = float(np.sqrt(3.0))
_S15 = float(np.sqrt(15.0))
_C22 = float(np.sqrt(5.0) / 2.0)
_C24 = float(np.sqrt(15.0) / 2.0)


def _rsqrt(r2):
    i = plsc.bitcast(r2, jnp.int32)
    i = 0x5F3759DF - lax.shift_right_logical(i, 1)
    y = plsc.bitcast(i, jnp.float32)
    for _ in range(3):
        y = y * (1.5 - 0.5 * r2 * y * y)
    return y


def _coef_body(posx, posy, posz, senders, receivers, coef_out,
               px, py, pz, sidx, ridx, cobuf):
    ci = lax.axis_index("c")
    si = lax.axis_index("s")
    wid = si * NC + ci
    base = wid * EPW
    pltpu.sync_copy(posx, px)
    pltpu.sync_copy(posy, py)
    pltpu.sync_copy(posz, pz)
    lane = lax.iota(jnp.int32, 16)

    def win(w, carry):
        wb = base + w * WA
        pltpu.sync_copy(senders.at[pl.ds(wb, WA)], sidx)
        pltpu.sync_copy(receivers.at[pl.ds(wb, WA)], ridx)

        def grp(g, carry2):
            off = jnp.minimum(g * 16, WA - 16)
            sv = sidx[pl.ds(off, 16)]
            rv = ridx[pl.ds(off, 16)]
            sx = plsc.load_gather(px, [sv])
            sy = plsc.load_gather(py, [sv])
            sz = plsc.load_gather(pz, [sv])
            rx = plsc.load_gather(px, [rv])
            ry = plsc.load_gather(py, [rv])
            rz = plsc.load_gather(pz, [rv])
            dx = rx - sx
            dy = ry - sy
            dz = rz - sz
            r2 = dx * dx + dy * dy + dz * dz + 1e-12
            y = _rsqrt(r2)
            ux = dx * y
            uy = dy * y
            uz = dz * y
            cs = [
                jnp.full((16,), 1.0, jnp.float32),
                _S3 * ux, _S3 * uy, _S3 * uz,
                _S15 * ux * uy, _S15 * uy * uz,
                _C22 * (3.0 * uz * uz - 1.0),
                _S15 * ux * uz, _C24 * (ux * ux - uy * uy),
            ]
            eidx = off + lane
            for k in range(9):
                kvec = jnp.full((16,), k, jnp.int32)
                plsc.store_scatter(cobuf, [eidx, kvec], cs[k])
            return carry2

        lax.fori_loop(0, (WA + 15) // 16, grp, 0)
        pltpu.sync_copy(cobuf, coef_out.at[pl.ds(wb, WA), :])
        return carry

    lax.fori_loop(0, EPW // WA, win, 0)


def _agg_body(nodes16, coef, senders, recv2d, agg_out,
              acc, sbuf, rbuf, gidx, xbuf, cbuf, ubuf, zbuf, sem):
    ci = lax.axis_index("c")
    si = lax.axis_index("s")

    def zrow(i, carry):
        for k in range(9):
            zbuf[i, pl.ds(k * 16, 16)] = jnp.zeros((16,), jnp.float32)
        return carry

    lax.fori_loop(0, ZROWS, zrow, 0)

    def qpass(q, carry):
        cglob = ci * CPS + q

        for j in range(ROWS_PT // ZROWS):
            pltpu.sync_copy(zbuf, acc.at[pl.ds(si * ROWS_PT + j * ZROWS, ZROWS), :])
        plsc.subcore_barrier()

        def win(w, carry2):
            eb = si * EPT + w * WB
            pltpu.sync_copy(senders.at[pl.ds(eb, WB)], sbuf)
            pltpu.sync_copy(recv2d.at[pl.ds(eb // SUB, NSUB), :], rbuf)

            def gi(j, carry3):
                def gg(g, carry4):
                    sv = sbuf[pl.ds(j * SUB + g * 16, 16)]
                    gidx[j, pl.ds(g * 16, 16)] = sv * 8 + cglob
                    return carry4
                lax.fori_loop(0, SUB // 16, gg, 0)
                return carry3

            lax.fori_loop(0, NSUB, gi, 0)

            for j in range(NSUB):
                pltpu.async_copy(nodes16.at[gidx.at[j]],
                                 xbuf.at[pl.ds(j * SUB, SUB), :], sem)
            for j in range(NSUB):
                pltpu.make_async_copy(nodes16.at[gidx.at[j]],
                                      xbuf.at[pl.ds(j * SUB, SUB), :], sem).wait()
            pltpu.sync_copy(coef.at[pl.ds(eb, WB), :], cbuf)

            def edge(e, carry3):
                x = xbuf[e, :]
                for k in range(9):
                    bk = plsc.load_gather(
                        cbuf, [jnp.full((16,), e, jnp.int32),
                               jnp.full((16,), k, jnp.int32)])
                    ubuf[e, pl.ds(k * 16, 16)] = x * bk
                return carry3

            lax.fori_loop(0, WB, edge, 0)

            for j in range(NSUB):
                pltpu.sync_copy(ubuf.at[pl.ds(j * SUB, SUB), :],
                                acc.at[rbuf.at[j]], add=True)
            return carry2

        lax.fori_loop(0, NWIN, win, 0)
        plsc.subcore_barrier()

        for k in range(9):
            pltpu.sync_copy(
                acc.at[pl.ds(si * ROWS_PT, ROWS_PT), pl.ds(k * 16, 16)],
                agg_out.at[k, pl.ds(si * ROWS_PT, ROWS_PT),
                           pl.ds(cglob * 16, 16)])
        plsc.subcore_barrier()
        return carry

    lax.fori_loop(0, CPS, qpass, 0)


def _prep_body(wpre1, wpost1, wpre2, wpost2, b1_ref, b2_ref):
    w1 = jnp.dot(wpre1[...], wpost1[...],
                 preferred_element_type=jnp.float32) * (INV * INV)
    w2 = jnp.dot(wpre2[...], wpost2[...],
                 preferred_element_type=jnp.float32) * (INV * INV)
    cols1 = lax.broadcasted_iota(jnp.int32, (D, 3 * D), 1)
    rows1 = lax.broadcasted_iota(jnp.int32, (D, 3 * D), 0)
    b1_parts = []
    for i in range(3):
        p = (cols1 == 3 * rows1 + i).astype(jnp.float32)
        b1_parts.append(jnp.dot(w1, p, preferred_element_type=jnp.float32))
    b1_ref[...] = jnp.concatenate(b1_parts, axis=0)
    cols2 = lax.broadcasted_iota(jnp.int32, (D, 5 * D), 1)
    rows2 = lax.broadcasted_iota(jnp.int32, (D, 5 * D), 0)
    b2_parts = []
    for i in range(5):
        p = (cols2 == 5 * rows2 + i).astype(jnp.float32)
        b2_parts.append(jnp.dot(w2, p, preferred_element_type=jnp.float32))
    b2_ref[...] = jnp.concatenate(b2_parts, axis=0)


def _main_body(agg, nodes_blk, wpre0, wpost0, wsc, b1, b2, out_ref):
    a = agg[...]
    s_agg = a[0] * (1.0 / DEN)
    h = jax.nn.gelu(jnp.dot(s_agg, wpre0[...],
                            preferred_element_type=jnp.float32) * INV)
    s_out = jnp.dot(h, wpost0[...], preferred_element_type=jnp.float32) * INV
    s_out = s_out + jnp.dot(nodes_blk[...], wsc[...],
                            preferred_element_type=jnp.float32) * INV
    cat1 = jnp.concatenate([a[1], a[2], a[3]], axis=1) * (1.0 / DEN)
    v1 = jnp.dot(cat1, b1[...], preferred_element_type=jnp.float32)
    cat2 = jnp.concatenate([a[4], a[5], a[6], a[7], a[8]], axis=1) * (1.0 / DEN)
    v2 = jnp.dot(cat2, b2[...], preferred_element_type=jnp.float32)
    out_ref[...] = jnp.concatenate([s_out, v1, v2], axis=1)


def kernel(nodes, positions, senders, receivers,
           W_pre0, W_pre1, W_pre2, W_post0, W_post1, W_post2, W_sc):
    senders = senders.astype(jnp.int32)
    receivers = receivers.astype(jnp.int32)
    posx = positions[:, 0]
    posy = positions[:, 1]
    posz = positions[:, 2]
    nodes16 = nodes.reshape(N * 8, 16)
    recv2d = receivers.reshape(E // SUB, SUB)

    mesh = plsc.VectorSubcoreMesh(core_axis_name="c", subcore_axis_name="s")

    sc_params = pltpu.CompilerParams(needs_layout_passes=False,
                                     use_tc_tiling_on_sc=False)

    coef = pl.kernel(
        _coef_body,
        mesh=mesh,
        compiler_params=sc_params,
        out_type=jax.ShapeDtypeStruct((E, 16), jnp.float32),
        scratch_types=[
            pltpu.VMEM((N,), jnp.float32),
            pltpu.VMEM((N,), jnp.float32),
            pltpu.VMEM((N,), jnp.float32),
            pltpu.VMEM((WA,), jnp.int32),
            pltpu.VMEM((WA,), jnp.int32),
            pltpu.VMEM((WA, 16), jnp.float32),
        ],
    )(posx, posy, posz, senders, receivers)

    agg = pl.kernel(
        _agg_body,
        mesh=mesh,
        compiler_params=sc_params,
        out_type=jax.ShapeDtypeStruct((9, N, D), jnp.float32),
        scratch_types=[
            pltpu.VMEM_SHARED((N, 144), jnp.float32),
            pltpu.VMEM((WB,), jnp.int32),
            pltpu.VMEM((NSUB, SUB), jnp.int32),
            pltpu.VMEM((NSUB, SUB), jnp.int32),
            pltpu.VMEM((WB, 16), jnp.float32),
            pltpu.VMEM((WB, 16), jnp.float32),
            pltpu.VMEM((WB, 144), jnp.float32),
            pltpu.VMEM((ZROWS, 144), jnp.float32),
            pltpu.SemaphoreType.DMA,
        ],
    )(nodes16, coef, senders, recv2d)

    b1, b2 = pl.pallas_call(
        _prep_body,
        out_shape=(jax.ShapeDtypeStruct((3 * D, 3 * D), jnp.float32),
                   jax.ShapeDtypeStruct((5 * D, 5 * D), jnp.float32)),
    )(W_pre1, W_post1, W_pre2, W_post2)

    BN = 1000
    out = pl.pallas_call(
        _main_body,
        grid=(N // BN,),
        in_specs=[
            pl.BlockSpec((9, BN, D), lambda i: (0, i, 0)),
            pl.BlockSpec((BN, D), lambda i: (i, 0)),
            pl.BlockSpec((D, D), lambda i: (0, 0)),
            pl.BlockSpec((D, D), lambda i: (0, 0)),
            pl.BlockSpec((D, D), lambda i: (0, 0)),
            pl.BlockSpec((3 * D, 3 * D), lambda i: (0, 0)),
            pl.BlockSpec((5 * D, 5 * D), lambda i: (0, 0)),
        ],
        out_specs=pl.BlockSpec((BN, 9 * D), lambda i: (i, 0)),
        out_shape=jax.ShapeDtypeStruct((N, 9 * D), jnp.float32),
    )(agg, nodes, W_pre0, W_post0, W_sc, b1, b2)
    return out


# speedup vs baseline: 3.5042x; 2.3391x over previous
import functools

import jax
import jax.numpy as jnp
import numpy as np
from jax import lax
from jax.experimental import pallas as pl
from jax.experimental.pallas import tpu as pltpu
from jax.experimental.pallas import tpu_sc as plsc

N = 10000
E = 160000
D = 128
DEN = 16.0
INV = 1.0 / np.sqrt(D)

NC = 2
NS = 16
NWORK = NC * NS

EPW = E // NWORK
WA = 1000
EPT = E // NS
WB = 80
SUB = 80
NSUB = WB // SUB
NWIN = EPT // WB
ROWS_PT = N // NS
ZROWS = 25
CPS = 4

_S3 = float(np.sqrt(3.0))
_S15 = float(np.sqrt(15.0))
_C22 = float(np.sqrt(5.0) / 2.0)
_C24 = float(np.sqrt(15.0) / 2.0)


def _rsqrt(r2):
    i = plsc.bitcast(r2, jnp.int32)
    i = 0x5F3759DF - lax.shift_right_logical(i, 1)
    y = plsc.bitcast(i, jnp.float32)
    for _ in range(3):
        y = y * (1.5 - 0.5 * r2 * y * y)
    return y


def _coef_body(posx, posy, posz, senders, receivers, coef_out,
               px, py, pz, sidx, ridx, cobuf):
    ci = lax.axis_index("c")
    si = lax.axis_index("s")
    wid = si * NC + ci
    base = wid * EPW
    pltpu.sync_copy(posx, px)
    pltpu.sync_copy(posy, py)
    pltpu.sync_copy(posz, pz)
    lane = lax.iota(jnp.int32, 16)

    def win(w, carry):
        wb = base + w * WA
        pltpu.sync_copy(senders.at[pl.ds(wb, WA)], sidx)
        pltpu.sync_copy(receivers.at[pl.ds(wb, WA)], ridx)

        def grp(g, carry2):
            off = jnp.minimum(g * 16, WA - 16)
            sv = sidx[pl.ds(off, 16)]
            rv = ridx[pl.ds(off, 16)]
            sx = plsc.load_gather(px, [sv])
            sy = plsc.load_gather(py, [sv])
            sz = plsc.load_gather(pz, [sv])
            rx = plsc.load_gather(px, [rv])
            ry = plsc.load_gather(py, [rv])
            rz = plsc.load_gather(pz, [rv])
            dx = rx - sx
            dy = ry - sy
            dz = rz - sz
            r2 = dx * dx + dy * dy + dz * dz + 1e-12
            y = _rsqrt(r2)
            ux = dx * y
            uy = dy * y
            uz = dz * y
            cs = [
                jnp.full((16,), 1.0, jnp.float32),
                _S3 * ux, _S3 * uy, _S3 * uz,
                _S15 * ux * uy, _S15 * uy * uz,
                _C22 * (3.0 * uz * uz - 1.0),
                _S15 * ux * uz, _C24 * (ux * ux - uy * uy),
            ]
            eidx = off + lane
            for k in range(9):
                kvec = jnp.full((16,), k, jnp.int32)
                plsc.store_scatter(cobuf, [eidx, kvec], cs[k])
            plsc.store_scatter(cobuf, [eidx, jnp.full((16,), 9, jnp.int32)],
                               plsc.bitcast(sv * 8, jnp.float32))
            plsc.store_scatter(cobuf, [eidx, jnp.full((16,), 10, jnp.int32)],
                               plsc.bitcast(rv, jnp.float32))
            return carry2

        lax.fori_loop(0, (WA + 15) // 16, grp, 0)
        pltpu.sync_copy(cobuf, coef_out.at[pl.ds(wb, WA), :])
        return carry

    lax.fori_loop(0, EPW // WA, win, 0)


def _agg_body(nodes16, coef, agg_out,
              acc, cb, xb, ub, gix, rbx, zbuf, sem_e, sem_x, sem_u):
    ci = lax.axis_index("c")
    si = lax.axis_index("s")
    lane = lax.iota(jnp.int32, 16)

    def zrow(i, carry):
        for k in range(9):
            zbuf[i, pl.ds(k * 16, 16)] = jnp.zeros((16,), jnp.float32)
        return carry

    lax.fori_loop(0, ZROWS, zrow, 0)

    def fire_erec(w, half):
        eb = si * EPT + w * WB
        pltpu.async_copy(coef.at[pl.ds(eb, WB), :],
                         cb.at[pl.ds(half * WB, WB), :], sem_e)

    def build_and_fire_gather(half, cglob):
        hb = half * WB
        for g in range(WB // 16):
            rows = hb + g * 16 + lane
            sv8 = plsc.bitcast(
                plsc.load_gather(cb, [rows, jnp.full((16,), 9, jnp.int32)]),
                jnp.int32)
            gix[half, pl.ds(g * 16, 16)] = sv8 + cglob
            rv = plsc.bitcast(
                plsc.load_gather(cb, [rows, jnp.full((16,), 10, jnp.int32)]),
                jnp.int32)
            rbx[half, pl.ds(g * 16, 16)] = rv
        pltpu.async_copy(nodes16.at[gix.at[half]],
                         xb.at[pl.ds(hb, WB), :], sem_x)

    def compute(half):
        hb = half * WB

        @plsc.parallel_loop(0, WB, unroll=4)
        def edge(e):
            pe = hb + e
            x = xb[pe, :]
            evec = jnp.full((16,), pe, jnp.int32)
            for k in range(9):
                bk = plsc.load_gather(
                    cb, [evec, jnp.full((16,), k, jnp.int32)])
                ub[pe, pl.ds(k * 16, 16)] = x * bk

    def fire_scatter(half):
        pltpu.async_copy(ub.at[pl.ds(half * WB, WB), :],
                         acc.at[rbx.at[half]], sem_u, add=True)

    def qpass(q, carry):
        cglob = ci * CPS + q

        for j in range(ROWS_PT // ZROWS):
            pltpu.sync_copy(zbuf, acc.at[pl.ds(si * ROWS_PT + j * ZROWS, ZROWS), :])
        plsc.subcore_barrier()

        def win(w, carry2):
            fire_erec(w, 0)
            pltpu.make_async_copy(
                coef.at[pl.ds(0, WB), :], cb.at[pl.ds(0, WB), :],
                sem_e).wait()
            build_and_fire_gather(0, cglob)
            pltpu.make_async_copy(
                nodes16.at[gix.at[0]], xb.at[pl.ds(0, WB), :],
                sem_x).wait()
            compute(0)
            pltpu.sync_copy(ub.at[pl.ds(0, WB), :], acc.at[rbx.at[0]],
                            add=True)
            return carry2

        lax.fori_loop(0, NWIN, win, 0)

        plsc.subcore_barrier()

        for k in range(9):
            pltpu.sync_copy(
                acc.at[pl.ds(si * ROWS_PT, ROWS_PT), pl.ds(k * 16, 16)],
                agg_out.at[k, pl.ds(si * ROWS_PT, ROWS_PT),
                           pl.ds(cglob * 16, 16)])
        plsc.subcore_barrier()
        return carry

    lax.fori_loop(0, CPS, qpass, 0)


def _prep_body(wpre1, wpost1, wpre2, wpost2, b1_ref, b2_ref):
    w1 = jnp.dot(wpre1[...], wpost1[...],
                 preferred_element_type=jnp.float32) * (INV * INV)
    w2 = jnp.dot(wpre2[...], wpost2[...],
                 preferred_element_type=jnp.float32) * (INV * INV)
    cols1 = lax.broadcasted_iota(jnp.int32, (D, 3 * D), 1)
    rows1 = lax.broadcasted_iota(jnp.int32, (D, 3 * D), 0)
    b1_parts = []
    for i in range(3):
        p = (cols1 == 3 * rows1 + i).astype(jnp.float32)
        b1_parts.append(jnp.dot(w1, p, preferred_element_type=jnp.float32))
    b1_ref[...] = jnp.concatenate(b1_parts, axis=0)
    cols2 = lax.broadcasted_iota(jnp.int32, (D, 5 * D), 1)
    rows2 = lax.broadcasted_iota(jnp.int32, (D, 5 * D), 0)
    b2_parts = []
    for i in range(5):
        p = (cols2 == 5 * rows2 + i).astype(jnp.float32)
        b2_parts.append(jnp.dot(w2, p, preferred_element_type=jnp.float32))
    b2_ref[...] = jnp.concatenate(b2_parts, axis=0)


def _main_body(agg, nodes_blk, wpre0, wpost0, wsc, b1, b2, out_ref):
    a = agg[...]
    s_agg = a[0] * (1.0 / DEN)
    h = jax.nn.gelu(jnp.dot(s_agg, wpre0[...],
                            preferred_element_type=jnp.float32) * INV)
    s_out = jnp.dot(h, wpost0[...], preferred_element_type=jnp.float32) * INV
    s_out = s_out + jnp.dot(nodes_blk[...], wsc[...],
                            preferred_element_type=jnp.float32) * INV
    cat1 = jnp.concatenate([a[1], a[2], a[3]], axis=1) * (1.0 / DEN)
    v1 = jnp.dot(cat1, b1[...], preferred_element_type=jnp.float32)
    cat2 = jnp.concatenate([a[4], a[5], a[6], a[7], a[8]], axis=1) * (1.0 / DEN)
    v2 = jnp.dot(cat2, b2[...], preferred_element_type=jnp.float32)
    out_ref[...] = jnp.concatenate([s_out, v1, v2], axis=1)


def kernel(nodes, positions, senders, receivers,
           W_pre0, W_pre1, W_pre2, W_post0, W_post1, W_post2, W_sc):
    senders = senders.astype(jnp.int32)
    receivers = receivers.astype(jnp.int32)
    posx = positions[:, 0]
    posy = positions[:, 1]
    posz = positions[:, 2]
    nodes16 = nodes.reshape(N * 8, 16)

    mesh = plsc.VectorSubcoreMesh(core_axis_name="c", subcore_axis_name="s")

    sc_params = pltpu.CompilerParams(needs_layout_passes=False,
                                     use_tc_tiling_on_sc=False)

    coef = pl.kernel(
        _coef_body,
        mesh=mesh,
        compiler_params=sc_params,
        out_type=jax.ShapeDtypeStruct((E, 16), jnp.float32),
        scratch_types=[
            pltpu.VMEM((N,), jnp.float32),
            pltpu.VMEM((N,), jnp.float32),
            pltpu.VMEM((N,), jnp.float32),
            pltpu.VMEM((WA,), jnp.int32),
            pltpu.VMEM((WA,), jnp.int32),
            pltpu.VMEM((WA, 16), jnp.float32),
        ],
    )(posx, posy, posz, senders, receivers)

    agg = pl.kernel(
        _agg_body,
        mesh=mesh,
        compiler_params=sc_params,
        out_type=jax.ShapeDtypeStruct((9, N, D), jnp.float32),
        scratch_types=[
            pltpu.VMEM_SHARED((N, 144), jnp.float32),
            pltpu.VMEM((2 * WB, 16), jnp.float32),
            pltpu.VMEM((2 * WB, 16), jnp.float32),
            pltpu.VMEM((2 * WB, 144), jnp.float32),
            pltpu.VMEM((2, WB), jnp.int32),
            pltpu.VMEM((2, WB), jnp.int32),
            pltpu.VMEM((ZROWS, 144), jnp.float32),
            pltpu.SemaphoreType.DMA,
            pltpu.SemaphoreType.DMA,
            pltpu.SemaphoreType.DMA,
        ],
    )(nodes16, coef)

    b1, b2 = pl.pallas_call(
        _prep_body,
        out_shape=(jax.ShapeDtypeStruct((3 * D, 3 * D), jnp.float32),
                   jax.ShapeDtypeStruct((5 * D, 5 * D), jnp.float32)),
    )(W_pre1, W_post1, W_pre2, W_post2)

    BN = 1000
    out = pl.pallas_call(
        _main_body,
        grid=(N // BN,),
        in_specs=[
            pl.BlockSpec((9, BN, D), lambda i: (0, i, 0)),
            pl.BlockSpec((BN, D), lambda i: (i, 0)),
            pl.BlockSpec((D, D), lambda i: (0, 0)),
            pl.BlockSpec((D, D), lambda i: (0, 0)),
            pl.BlockSpec((D, D), lambda i: (0, 0)),
            pl.BlockSpec((3 * D, 3 * D), lambda i: (0, 0)),
            pl.BlockSpec((5 * D, 5 * D), lambda i: (0, 0)),
        ],
        out_specs=pl.BlockSpec((BN, 9 * D), lambda i: (i, 0)),
        out_shape=jax.ShapeDtypeStruct((N, 9 * D), jnp.float32),
    )(agg, nodes, W_pre0, W_post0, W_sc, b1, b2)
    return out


# speedup vs baseline: 5.6380x; 1.6089x over previous
import functools

import jax
import jax.numpy as jnp
import numpy as np
from jax import lax
from jax.experimental import pallas as pl
from jax.experimental.pallas import tpu as pltpu
from jax.experimental.pallas import tpu_sc as plsc

N = 10000
E = 160000
D = 128
DEN = 16.0
INV = 1.0 / np.sqrt(D)

NC = 2
NS = 16
NWORK = NC * NS

EPW = E // NWORK
WA = 1000
EPT = E // NS
WB = 80
SUB = 80
NSUB = WB // SUB
NWIN = EPT // WB
ROWS_PT = N // NS
ZROWS = 25
CPS = 4

_S3 = float(np.sqrt(3.0))
_S15 = float(np.sqrt(15.0))
_C22 = float(np.sqrt(5.0) / 2.0)
_C24 = float(np.sqrt(15.0) / 2.0)


def _rsqrt(r2):
    i = plsc.bitcast(r2, jnp.int32)
    i = 0x5F3759DF - lax.shift_right_logical(i, 1)
    y = plsc.bitcast(i, jnp.float32)
    for _ in range(3):
        y = y * (1.5 - 0.5 * r2 * y * y)
    return y


def _coef_body(posx, posy, posz, senders, receivers, coef_out,
               px, py, pz, sidx, ridx, cobuf):
    ci = lax.axis_index("c")
    si = lax.axis_index("s")
    wid = si * NC + ci
    base = wid * EPW
    pltpu.sync_copy(posx, px)
    pltpu.sync_copy(posy, py)
    pltpu.sync_copy(posz, pz)
    lane = lax.iota(jnp.int32, 16)

    def win(w, carry):
        wb = base + w * WA
        pltpu.sync_copy(senders.at[pl.ds(wb, WA)], sidx)
        pltpu.sync_copy(receivers.at[pl.ds(wb, WA)], ridx)

        def grp(g, carry2):
            off = jnp.minimum(g * 16, WA - 16)
            sv = sidx[pl.ds(off, 16)]
            rv = ridx[pl.ds(off, 16)]
            sx = plsc.load_gather(px, [sv])
            sy = plsc.load_gather(py, [sv])
            sz = plsc.load_gather(pz, [sv])
            rx = plsc.load_gather(px, [rv])
            ry = plsc.load_gather(py, [rv])
            rz = plsc.load_gather(pz, [rv])
            dx = rx - sx
            dy = ry - sy
            dz = rz - sz
            r2 = dx * dx + dy * dy + dz * dz + 1e-12
            y = _rsqrt(r2)
            ux = dx * y
            uy = dy * y
            uz = dz * y
            cs = [
                jnp.full((16,), 1.0, jnp.float32),
                _S3 * ux, _S3 * uy, _S3 * uz,
                _S15 * ux * uy, _S15 * uy * uz,
                _C22 * (3.0 * uz * uz - 1.0),
                _S15 * ux * uz, _C24 * (ux * ux - uy * uy),
            ]
            eidx = off + lane
            for k in range(9):
                kvec = jnp.full((16,), k, jnp.int32)
                plsc.store_scatter(cobuf, [eidx, kvec], cs[k])
            plsc.store_scatter(cobuf, [eidx, jnp.full((16,), 9, jnp.int32)],
                               plsc.bitcast(sv * 8, jnp.float32))
            plsc.store_scatter(cobuf, [eidx, jnp.full((16,), 10, jnp.int32)],
                               plsc.bitcast(rv, jnp.float32))
            return carry2

        lax.fori_loop(0, (WA + 15) // 16, grp, 0)
        pltpu.sync_copy(cobuf, coef_out.at[pl.ds(wb, WA), :])
        return carry

    lax.fori_loop(0, EPW // WA, win, 0)


def _agg_body(nodes16, coef, agg_out,
              acc, cb, xb, ub, gix, rbx, zbuf, sem_e, sem_x, sem_u):
    ci = lax.axis_index("c")
    si = lax.axis_index("s")
    lane = lax.iota(jnp.int32, 16)

    def zrow(i, carry):
        for k in range(9):
            zbuf[i, pl.ds(k * 16, 16)] = jnp.zeros((16,), jnp.float32)
        return carry

    lax.fori_loop(0, ZROWS, zrow, 0)

    def fire_erec(w, half):
        eb = si * EPT + w * WB
        pltpu.async_copy(coef.at[pl.ds(eb, WB), :],
                         cb.at[pl.ds(half * WB, WB), :], sem_e)

    def build_and_fire_gather(half, cglob):
        hb = half * WB
        for g in range(WB // 16):
            rows = hb + g * 16 + lane
            sv8 = plsc.bitcast(
                plsc.load_gather(cb, [rows, jnp.full((16,), 9, jnp.int32)]),
                jnp.int32)
            gix[half, pl.ds(g * 16, 16)] = sv8 + cglob
            rv = plsc.bitcast(
                plsc.load_gather(cb, [rows, jnp.full((16,), 10, jnp.int32)]),
                jnp.int32)
            rbx[half, pl.ds(g * 16, 16)] = rv
        pltpu.async_copy(nodes16.at[gix.at[half]],
                         xb.at[pl.ds(hb, WB), :], sem_x)

    def compute(xq, uh):
        xbase = xq * WB
        ubase = uh * WB

        @plsc.parallel_loop(0, WB, unroll=4)
        def edge(e):
            x = xb[xbase + e, :]
            evec = jnp.full((16,), xbase + e, jnp.int32)
            for k in range(9):
                bk = plsc.load_gather(
                    cb, [evec, jnp.full((16,), k, jnp.int32)])
                ub[ubase + e, pl.ds(k * 16, 16)] = x * bk

    def fire_scatter(xq, uh):
        pltpu.async_copy(ub.at[pl.ds(uh * WB, WB), :],
                         acc.at[rbx.at[xq]], sem_u, add=True)

    def drain_scatter(xq, uh):
        pltpu.make_async_copy(ub.at[pl.ds(uh * WB, WB), :],
                              acc.at[rbx.at[xq]], sem_u).wait()

    def qpass(q, carry):
        cglob = ci * CPS + q

        for j in range(ROWS_PT // ZROWS):
            pltpu.sync_copy(zbuf, acc.at[pl.ds(si * ROWS_PT + j * ZROWS, ZROWS), :])
        plsc.subcore_barrier()

        def win4(t, carry2):
            w0 = t * 4
            for h in range(4):
                fire_erec(w0 + h, h)
            for h in range(4):
                pltpu.make_async_copy(
                    coef.at[pl.ds(0, WB), :], cb.at[pl.ds(h * WB, WB), :],
                    sem_e).wait()
            for h in range(4):
                build_and_fire_gather(h, cglob)
            for h in range(4):
                pltpu.make_async_copy(
                    nodes16.at[gix.at[h]], xb.at[pl.ds(h * WB, WB), :],
                    sem_x).wait()
            compute(0, 0)
            fire_scatter(0, 0)
            compute(1, 1)
            fire_scatter(1, 1)
            drain_scatter(0, 0)
            drain_scatter(1, 1)
            compute(2, 0)
            fire_scatter(2, 0)
            compute(3, 1)
            fire_scatter(3, 1)
            drain_scatter(2, 0)
            drain_scatter(3, 1)
            return carry2

        lax.fori_loop(0, NWIN // 4, win4, 0)

        fire_erec(NWIN - 1, 0)
        pltpu.make_async_copy(coef.at[pl.ds(0, WB), :],
                              cb.at[pl.ds(0, WB), :], sem_e).wait()
        build_and_fire_gather(0, cglob)
        pltpu.make_async_copy(nodes16.at[gix.at[0]],
                              xb.at[pl.ds(0, WB), :], sem_x).wait()
        compute(0, 0)
        pltpu.sync_copy(ub.at[pl.ds(0, WB), :], acc.at[rbx.at[0]], add=True)

        plsc.subcore_barrier()

        for k in range(9):
            pltpu.sync_copy(
                acc.at[pl.ds(si * ROWS_PT, ROWS_PT), pl.ds(k * 16, 16)],
                agg_out.at[k, pl.ds(si * ROWS_PT, ROWS_PT),
                           pl.ds(cglob * 16, 16)])
        plsc.subcore_barrier()
        return carry

    lax.fori_loop(0, CPS, qpass, 0)


def _prep_body(wpre1, wpost1, wpre2, wpost2, b1_ref, b2_ref):
    w1 = jnp.dot(wpre1[...], wpost1[...],
                 preferred_element_type=jnp.float32) * (INV * INV)
    w2 = jnp.dot(wpre2[...], wpost2[...],
                 preferred_element_type=jnp.float32) * (INV * INV)
    cols1 = lax.broadcasted_iota(jnp.int32, (D, 3 * D), 1)
    rows1 = lax.broadcasted_iota(jnp.int32, (D, 3 * D), 0)
    b1_parts = []
    for i in range(3):
        p = (cols1 == 3 * rows1 + i).astype(jnp.float32)
        b1_parts.append(jnp.dot(w1, p, preferred_element_type=jnp.float32))
    b1_ref[...] = jnp.concatenate(b1_parts, axis=0)
    cols2 = lax.broadcasted_iota(jnp.int32, (D, 5 * D), 1)
    rows2 = lax.broadcasted_iota(jnp.int32, (D, 5 * D), 0)
    b2_parts = []
    for i in range(5):
        p = (cols2 == 5 * rows2 + i).astype(jnp.float32)
        b2_parts.append(jnp.dot(w2, p, preferred_element_type=jnp.float32))
    b2_ref[...] = jnp.concatenate(b2_parts, axis=0)


def _main_body(agg, nodes_blk, wpre0, wpost0, wsc, b1, b2, out_ref):
    a = agg[...]
    s_agg = a[0] * (1.0 / DEN)
    h = jax.nn.gelu(jnp.dot(s_agg, wpre0[...],
                            preferred_element_type=jnp.float32) * INV)
    s_out = jnp.dot(h, wpost0[...], preferred_element_type=jnp.float32) * INV
    s_out = s_out + jnp.dot(nodes_blk[...], wsc[...],
                            preferred_element_type=jnp.float32) * INV
    cat1 = jnp.concatenate([a[1], a[2], a[3]], axis=1) * (1.0 / DEN)
    v1 = jnp.dot(cat1, b1[...], preferred_element_type=jnp.float32)
    cat2 = jnp.concatenate([a[4], a[5], a[6], a[7], a[8]], axis=1) * (1.0 / DEN)
    v2 = jnp.dot(cat2, b2[...], preferred_element_type=jnp.float32)
    out_ref[...] = jnp.concatenate([s_out, v1, v2], axis=1)


def kernel(nodes, positions, senders, receivers,
           W_pre0, W_pre1, W_pre2, W_post0, W_post1, W_post2, W_sc):
    senders = senders.astype(jnp.int32)
    receivers = receivers.astype(jnp.int32)
    posx = positions[:, 0]
    posy = positions[:, 1]
    posz = positions[:, 2]
    nodes16 = nodes.reshape(N * 8, 16)

    mesh = plsc.VectorSubcoreMesh(core_axis_name="c", subcore_axis_name="s")

    sc_params = pltpu.CompilerParams(needs_layout_passes=False,
                                     use_tc_tiling_on_sc=False)

    coef = pl.kernel(
        _coef_body,
        mesh=mesh,
        compiler_params=sc_params,
        out_type=jax.ShapeDtypeStruct((E, 16), jnp.float32),
        scratch_types=[
            pltpu.VMEM((N,), jnp.float32),
            pltpu.VMEM((N,), jnp.float32),
            pltpu.VMEM((N,), jnp.float32),
            pltpu.VMEM((WA,), jnp.int32),
            pltpu.VMEM((WA,), jnp.int32),
            pltpu.VMEM((WA, 16), jnp.float32),
        ],
    )(posx, posy, posz, senders, receivers)

    agg = pl.kernel(
        _agg_body,
        mesh=mesh,
        compiler_params=sc_params,
        out_type=jax.ShapeDtypeStruct((9, N, D), jnp.float32),
        scratch_types=[
            pltpu.VMEM_SHARED((N, 144), jnp.float32),
            pltpu.VMEM((4 * WB, 16), jnp.float32),
            pltpu.VMEM((4 * WB, 16), jnp.float32),
            pltpu.VMEM((2 * WB, 144), jnp.float32),
            pltpu.VMEM((4, WB), jnp.int32),
            pltpu.VMEM((4, WB), jnp.int32),
            pltpu.VMEM((ZROWS, 144), jnp.float32),
            pltpu.SemaphoreType.DMA,
            pltpu.SemaphoreType.DMA,
            pltpu.SemaphoreType.DMA,
        ],
    )(nodes16, coef)

    b1, b2 = pl.pallas_call(
        _prep_body,
        out_shape=(jax.ShapeDtypeStruct((3 * D, 3 * D), jnp.float32),
                   jax.ShapeDtypeStruct((5 * D, 5 * D), jnp.float32)),
    )(W_pre1, W_post1, W_pre2, W_post2)

    BN = 1000
    out = pl.pallas_call(
        _main_body,
        grid=(N // BN,),
        in_specs=[
            pl.BlockSpec((9, BN, D), lambda i: (0, i, 0)),
            pl.BlockSpec((BN, D), lambda i: (i, 0)),
            pl.BlockSpec((D, D), lambda i: (0, 0)),
            pl.BlockSpec((D, D), lambda i: (0, 0)),
            pl.BlockSpec((D, D), lambda i: (0, 0)),
            pl.BlockSpec((3 * D, 3 * D), lambda i: (0, 0)),
            pl.BlockSpec((5 * D, 5 * D), lambda i: (0, 0)),
        ],
        out_specs=pl.BlockSpec((BN, 9 * D), lambda i: (i, 0)),
        out_shape=jax.ShapeDtypeStruct((N, 9 * D), jnp.float32),
    )(agg, nodes, W_pre0, W_post0, W_sc, b1, b2)
    return out


# speedup vs baseline: 5.9901x; 1.0624x over previous
import functools

import jax
import jax.numpy as jnp
import numpy as np
from jax import lax
from jax.experimental import pallas as pl
from jax.experimental.pallas import tpu as pltpu
from jax.experimental.pallas import tpu_sc as plsc

N = 10000
E = 160000
D = 128
DEN = 16.0
INV = 1.0 / np.sqrt(D)

NC = 2
NS = 16
NWORK = NC * NS

EPW = E // NWORK
WA = 1000
EPT = E // NS
WB = 80
SUB = 80
NSUB = WB // SUB
NWIN = EPT // WB
ROWS_PT = N // NS
ZROWS = 25
CPS = 4

_S3 = float(np.sqrt(3.0))
_S15 = float(np.sqrt(15.0))
_C22 = float(np.sqrt(5.0) / 2.0)
_C24 = float(np.sqrt(15.0) / 2.0)


def _rsqrt(r2):
    i = plsc.bitcast(r2, jnp.int32)
    i = 0x5F3759DF - lax.shift_right_logical(i, 1)
    y = plsc.bitcast(i, jnp.float32)
    for _ in range(3):
        y = y * (1.5 - 0.5 * r2 * y * y)
    return y


def _coef_body(posx, posy, posz, senders, receivers, coef_out,
               px, py, pz, sidx, ridx, cobuf):
    ci = lax.axis_index("c")
    si = lax.axis_index("s")
    wid = si * NC + ci
    base = wid * EPW
    pltpu.sync_copy(posx, px)
    pltpu.sync_copy(posy, py)
    pltpu.sync_copy(posz, pz)
    lane = lax.iota(jnp.int32, 16)

    def win(w, carry):
        wb = base + w * WA
        pltpu.sync_copy(senders.at[pl.ds(wb, WA)], sidx)
        pltpu.sync_copy(receivers.at[pl.ds(wb, WA)], ridx)

        def grp(g, carry2):
            off = jnp.minimum(g * 16, WA - 16)
            sv = sidx[pl.ds(off, 16)]
            rv = ridx[pl.ds(off, 16)]
            sx = plsc.load_gather(px, [sv])
            sy = plsc.load_gather(py, [sv])
            sz = plsc.load_gather(pz, [sv])
            rx = plsc.load_gather(px, [rv])
            ry = plsc.load_gather(py, [rv])
            rz = plsc.load_gather(pz, [rv])
            dx = rx - sx
            dy = ry - sy
            dz = rz - sz
            r2 = dx * dx + dy * dy + dz * dz + 1e-12
            y = _rsqrt(r2)
            ux = dx * y
            uy = dy * y
            uz = dz * y
            cs = [
                jnp.full((16,), 1.0, jnp.float32),
                _S3 * ux, _S3 * uy, _S3 * uz,
                _S15 * ux * uy, _S15 * uy * uz,
                _C22 * (3.0 * uz * uz - 1.0),
                _S15 * ux * uz, _C24 * (ux * ux - uy * uy),
            ]
            eidx = off + lane
            for k in range(9):
                kvec = jnp.full((16,), k, jnp.int32)
                plsc.store_scatter(cobuf, [eidx, kvec], cs[k])
            plsc.store_scatter(cobuf, [eidx, jnp.full((16,), 9, jnp.int32)],
                               plsc.bitcast(sv * 8, jnp.float32))
            plsc.store_scatter(cobuf, [eidx, jnp.full((16,), 10, jnp.int32)],
                               plsc.bitcast(rv, jnp.float32))
            return carry2

        lax.fori_loop(0, (WA + 15) // 16, grp, 0)
        pltpu.sync_copy(cobuf, coef_out.at[pl.ds(wb, WA), :])
        return carry

    lax.fori_loop(0, EPW // WA, win, 0)


def _agg_body(nodes16, coef, agg_out,
              acc, cb, xb, ub, gix, rbx, zbuf, sem_e, sem_x, sem_u):
    ci = lax.axis_index("c")
    si = lax.axis_index("s")
    lane = lax.iota(jnp.int32, 16)

    def zrow(i, carry):
        for k in range(9):
            zbuf[i, pl.ds(k * 16, 16)] = jnp.zeros((16,), jnp.float32)
        return carry

    lax.fori_loop(0, ZROWS, zrow, 0)

    def fire_erec(w, half):
        eb = si * EPT + w * WB
        pltpu.async_copy(coef.at[pl.ds(eb, WB), :],
                         cb.at[pl.ds(half * WB, WB), :], sem_e)

    def build_and_fire_gather(half, cglob):
        hb = half * WB
        for g in range(WB // 16):
            rows = hb + g * 16 + lane
            sv8 = plsc.bitcast(
                plsc.load_gather(cb, [rows, jnp.full((16,), 9, jnp.int32)]),
                jnp.int32)
            gix[half, pl.ds(g * 16, 16)] = sv8 + cglob
            rv = plsc.bitcast(
                plsc.load_gather(cb, [rows, jnp.full((16,), 10, jnp.int32)]),
                jnp.int32)
            rbx[half, pl.ds(g * 16, 16)] = rv
        pltpu.async_copy(nodes16.at[gix.at[half]],
                         xb.at[pl.ds(hb, WB), :], sem_x)

    def compute(xq, uh):
        xbase = xq * WB
        ubase = uh * WB

        @plsc.parallel_loop(0, WB, unroll=8)
        def edge(e):
            x = xb[xbase + e, :]
            evec = jnp.full((16,), xbase + e, jnp.int32)
            for k in range(9):
                bk = plsc.load_gather(
                    cb, [evec, jnp.full((16,), k, jnp.int32)])
                ub[ubase + e, pl.ds(k * 16, 16)] = x * bk

    def fire_scatter(xq, uh):
        pltpu.async_copy(ub.at[pl.ds(uh * WB, WB), :],
                         acc.at[rbx.at[xq]], sem_u, add=True)

    def drain_scatter(xq, uh):
        pltpu.make_async_copy(ub.at[pl.ds(uh * WB, WB), :],
                              acc.at[rbx.at[xq]], sem_u).wait()

    def qpass(q, carry):
        cglob = ci * CPS + q

        for j in range(ROWS_PT // ZROWS):
            pltpu.async_copy(
                zbuf, acc.at[pl.ds(si * ROWS_PT + j * ZROWS, ZROWS), :],
                sem_e)
        for j in range(ROWS_PT // ZROWS):
            pltpu.make_async_copy(
                zbuf, acc.at[pl.ds(si * ROWS_PT + j * ZROWS, ZROWS), :],
                sem_e).wait()
        plsc.subcore_barrier()

        def win4(t, carry2):
            w0 = t * 4
            for h in range(4):
                fire_erec(w0 + h, h)
            for h in range(4):
                pltpu.make_async_copy(
                    coef.at[pl.ds(0, WB), :], cb.at[pl.ds(h * WB, WB), :],
                    sem_e).wait()
            for h in range(4):
                build_and_fire_gather(h, cglob)
            for h in range(4):
                pltpu.make_async_copy(
                    nodes16.at[gix.at[h]], xb.at[pl.ds(h * WB, WB), :],
                    sem_x).wait()
            compute(0, 0)
            fire_scatter(0, 0)
            compute(1, 1)
            fire_scatter(1, 1)
            drain_scatter(0, 0)
            drain_scatter(1, 1)
            compute(2, 0)
            fire_scatter(2, 0)
            compute(3, 1)
            fire_scatter(3, 1)
            drain_scatter(2, 0)
            drain_scatter(3, 1)
            return carry2

        lax.fori_loop(0, NWIN // 4, win4, 0)

        fire_erec(NWIN - 1, 0)
        pltpu.make_async_copy(coef.at[pl.ds(0, WB), :],
                              cb.at[pl.ds(0, WB), :], sem_e).wait()
        build_and_fire_gather(0, cglob)
        pltpu.make_async_copy(nodes16.at[gix.at[0]],
                              xb.at[pl.ds(0, WB), :], sem_x).wait()
        compute(0, 0)
        pltpu.sync_copy(ub.at[pl.ds(0, WB), :], acc.at[rbx.at[0]], add=True)

        plsc.subcore_barrier()

        for k in range(9):
            pltpu.async_copy(
                acc.at[pl.ds(si * ROWS_PT, ROWS_PT), pl.ds(k * 16, 16)],
                agg_out.at[k, pl.ds(si * ROWS_PT, ROWS_PT),
                           pl.ds(cglob * 16, 16)], sem_e)
        for k in range(9):
            pltpu.make_async_copy(
                acc.at[pl.ds(si * ROWS_PT, ROWS_PT), pl.ds(k * 16, 16)],
                agg_out.at[k, pl.ds(si * ROWS_PT, ROWS_PT),
                           pl.ds(cglob * 16, 16)], sem_e).wait()
        plsc.subcore_barrier()
        return carry

    lax.fori_loop(0, CPS, qpass, 0)


def _prep_body(wpre1, wpost1, wpre2, wpost2, b1_ref, b2_ref):
    w1 = jnp.dot(wpre1[...], wpost1[...],
                 preferred_element_type=jnp.float32) * (INV * INV)
    w2 = jnp.dot(wpre2[...], wpost2[...],
                 preferred_element_type=jnp.float32) * (INV * INV)
    cols1 = lax.broadcasted_iota(jnp.int32, (D, 3 * D), 1)
    rows1 = lax.broadcasted_iota(jnp.int32, (D, 3 * D), 0)
    b1_parts = []
    for i in range(3):
        p = (cols1 == 3 * rows1 + i).astype(jnp.float32)
        b1_parts.append(jnp.dot(w1, p, preferred_element_type=jnp.float32))
    b1_ref[...] = jnp.concatenate(b1_parts, axis=0)
    cols2 = lax.broadcasted_iota(jnp.int32, (D, 5 * D), 1)
    rows2 = lax.broadcasted_iota(jnp.int32, (D, 5 * D), 0)
    b2_parts = []
    for i in range(5):
        p = (cols2 == 5 * rows2 + i).astype(jnp.float32)
        b2_parts.append(jnp.dot(w2, p, preferred_element_type=jnp.float32))
    b2_ref[...] = jnp.concatenate(b2_parts, axis=0)


def _main_body(agg, nodes_blk, wpre0, wpost0, wsc, b1, b2, out_ref):
    a = agg[...]
    s_agg = a[0] * (1.0 / DEN)
    h = jax.nn.gelu(jnp.dot(s_agg, wpre0[...],
                            preferred_element_type=jnp.float32) * INV)
    s_out = jnp.dot(h, wpost0[...], preferred_element_type=jnp.float32) * INV
    s_out = s_out + jnp.dot(nodes_blk[...], wsc[...],
                            preferred_element_type=jnp.float32) * INV
    cat1 = jnp.concatenate([a[1], a[2], a[3]], axis=1) * (1.0 / DEN)
    v1 = jnp.dot(cat1, b1[...], preferred_element_type=jnp.float32)
    cat2 = jnp.concatenate([a[4], a[5], a[6], a[7], a[8]], axis=1) * (1.0 / DEN)
    v2 = jnp.dot(cat2, b2[...], preferred_element_type=jnp.float32)
    out_ref[...] = jnp.concatenate([s_out, v1, v2], axis=1)


def kernel(nodes, positions, senders, receivers,
           W_pre0, W_pre1, W_pre2, W_post0, W_post1, W_post2, W_sc):
    senders = senders.astype(jnp.int32)
    receivers = receivers.astype(jnp.int32)
    posx = positions[:, 0]
    posy = positions[:, 1]
    posz = positions[:, 2]
    nodes16 = nodes.reshape(N * 8, 16)

    mesh = plsc.VectorSubcoreMesh(core_axis_name="c", subcore_axis_name="s")

    sc_params = pltpu.CompilerParams(needs_layout_passes=False,
                                     use_tc_tiling_on_sc=False)

    coef = pl.kernel(
        _coef_body,
        mesh=mesh,
        compiler_params=sc_params,
        out_type=jax.ShapeDtypeStruct((E, 16), jnp.float32),
        scratch_types=[
            pltpu.VMEM((N,), jnp.float32),
            pltpu.VMEM((N,), jnp.float32),
            pltpu.VMEM((N,), jnp.float32),
            pltpu.VMEM((WA,), jnp.int32),
            pltpu.VMEM((WA,), jnp.int32),
            pltpu.VMEM((WA, 16), jnp.float32),
        ],
    )(posx, posy, posz, senders, receivers)

    agg = pl.kernel(
        _agg_body,
        mesh=mesh,
        compiler_params=sc_params,
        out_type=jax.ShapeDtypeStruct((9, N, D), jnp.float32),
        scratch_types=[
            pltpu.VMEM_SHARED((N, 144), jnp.float32),
            pltpu.VMEM((4 * WB, 16), jnp.float32),
            pltpu.VMEM((4 * WB, 16), jnp.float32),
            pltpu.VMEM((2 * WB, 144), jnp.float32),
            pltpu.VMEM((4, WB), jnp.int32),
            pltpu.VMEM((4, WB), jnp.int32),
            pltpu.VMEM((ZROWS, 144), jnp.float32),
            pltpu.SemaphoreType.DMA,
            pltpu.SemaphoreType.DMA,
            pltpu.SemaphoreType.DMA,
        ],
    )(nodes16, coef)

    b1, b2 = pl.pallas_call(
        _prep_body,
        out_shape=(jax.ShapeDtypeStruct((3 * D, 3 * D), jnp.float32),
                   jax.ShapeDtypeStruct((5 * D, 5 * D), jnp.float32)),
    )(W_pre1, W_post1, W_pre2, W_post2)

    BN = 1000
    out = pl.pallas_call(
        _main_body,
        grid=(N // BN,),
        in_specs=[
            pl.BlockSpec((9, BN, D), lambda i: (0, i, 0)),
            pl.BlockSpec((BN, D), lambda i: (i, 0)),
            pl.BlockSpec((D, D), lambda i: (0, 0)),
            pl.BlockSpec((D, D), lambda i: (0, 0)),
            pl.BlockSpec((D, D), lambda i: (0, 0)),
            pl.BlockSpec((3 * D, 3 * D), lambda i: (0, 0)),
            pl.BlockSpec((5 * D, 5 * D), lambda i: (0, 0)),
        ],
        out_specs=pl.BlockSpec((BN, 9 * D), lambda i: (i, 0)),
        out_shape=jax.ShapeDtypeStruct((N, 9 * D), jnp.float32),
    )(agg, nodes, W_pre0, W_post0, W_sc, b1, b2)
    return out


# speedup vs baseline: 6.2071x; 1.0362x over previous
import functools

import jax
import jax.numpy as jnp
import numpy as np
from jax import lax
from jax.experimental import pallas as pl
from jax.experimental.pallas import tpu as pltpu
from jax.experimental.pallas import tpu_sc as plsc

N = 10000
E = 160000
D = 128
DEN = 16.0
INV = 1.0 / np.sqrt(D)

NC = 2
NS = 16
NWORK = NC * NS

EPW = E // NWORK
WA = 1000
EPT = E // NS
WB = 80
SUB = 80
NSUB = WB // SUB
NWIN = EPT // WB
ROWS_PT = N // NS
ZROWS = 25
CPS = 4

_S3 = float(np.sqrt(3.0))
_S15 = float(np.sqrt(15.0))
_C22 = float(np.sqrt(5.0) / 2.0)
_C24 = float(np.sqrt(15.0) / 2.0)


def _rsqrt(r2):
    i = plsc.bitcast(r2, jnp.int32)
    i = 0x5F3759DF - lax.shift_right_logical(i, 1)
    y = plsc.bitcast(i, jnp.float32)
    for _ in range(3):
        y = y * (1.5 - 0.5 * r2 * y * y)
    return y


def _coef_body(posx, posy, posz, senders, receivers, coef_out,
               px, py, pz, sidx, ridx, cobuf):
    ci = lax.axis_index("c")
    si = lax.axis_index("s")
    wid = si * NC + ci
    base = wid * EPW
    pltpu.sync_copy(posx, px)
    pltpu.sync_copy(posy, py)
    pltpu.sync_copy(posz, pz)
    lane = lax.iota(jnp.int32, 16)

    def win(w, carry):
        wb = base + w * WA
        pltpu.sync_copy(senders.at[pl.ds(wb, WA)], sidx)
        pltpu.sync_copy(receivers.at[pl.ds(wb, WA)], ridx)

        def grp(g, carry2):
            off = jnp.minimum(g * 16, WA - 16)
            sv = sidx[pl.ds(off, 16)]
            rv = ridx[pl.ds(off, 16)]
            sx = plsc.load_gather(px, [sv])
            sy = plsc.load_gather(py, [sv])
            sz = plsc.load_gather(pz, [sv])
            rx = plsc.load_gather(px, [rv])
            ry = plsc.load_gather(py, [rv])
            rz = plsc.load_gather(pz, [rv])
            dx = rx - sx
            dy = ry - sy
            dz = rz - sz
            r2 = dx * dx + dy * dy + dz * dz + 1e-12
            y = _rsqrt(r2)
            ux = dx * y
            uy = dy * y
            uz = dz * y
            cs = [
                jnp.full((16,), 1.0, jnp.float32),
                _S3 * ux, _S3 * uy, _S3 * uz,
                _S15 * ux * uy, _S15 * uy * uz,
                _C22 * (3.0 * uz * uz - 1.0),
                _S15 * ux * uz, _C24 * (ux * ux - uy * uy),
            ]
            eidx = off + lane
            for k in range(9):
                kvec = jnp.full((16,), k, jnp.int32)
                plsc.store_scatter(cobuf, [eidx, kvec], cs[k])
            plsc.store_scatter(cobuf, [eidx, jnp.full((16,), 9, jnp.int32)],
                               plsc.bitcast(sv * 8, jnp.float32))
            plsc.store_scatter(cobuf, [eidx, jnp.full((16,), 10, jnp.int32)],
                               plsc.bitcast(rv, jnp.float32))
            return carry2

        lax.fori_loop(0, (WA + 15) // 16, grp, 0)
        pltpu.sync_copy(cobuf, coef_out.at[pl.ds(wb, WA), :])
        return carry

    lax.fori_loop(0, EPW // WA, win, 0)


def _agg_body(nodes16, coef, agg_out,
              acc, cb, xb, ub, gix, rbx, zbuf, sem_e, sem_x, sem_u, sem_y):
    ci = lax.axis_index("c")
    si = lax.axis_index("s")
    lane = lax.iota(jnp.int32, 16)

    def zrow(i, carry):
        for k in range(9):
            zbuf[i, pl.ds(k * 16, 16)] = jnp.zeros((16,), jnp.float32)
        return carry

    lax.fori_loop(0, ZROWS, zrow, 0)

    def fire_erec(w, half):
        eb = si * EPT + w * WB
        pltpu.async_copy(coef.at[pl.ds(eb, WB), :],
                         cb.at[pl.ds(half * WB, WB), :], sem_e)

    def build_and_fire_gather(half, cglob, sem):
        hb = half * WB
        for g in range(WB // 16):
            rows = hb + g * 16 + lane
            sv8 = plsc.bitcast(
                plsc.load_gather(cb, [rows, jnp.full((16,), 9, jnp.int32)]),
                jnp.int32)
            gix[half, pl.ds(g * 16, 16)] = sv8 + cglob
            rv = plsc.bitcast(
                plsc.load_gather(cb, [rows, jnp.full((16,), 10, jnp.int32)]),
                jnp.int32)
            rbx[half, pl.ds(g * 16, 16)] = rv
        pltpu.async_copy(nodes16.at[gix.at[half]],
                         xb.at[pl.ds(hb, WB), :], sem)

    def compute(xq, uh):
        xbase = xq * WB
        ubase = uh * WB

        @plsc.parallel_loop(0, WB, unroll=8)
        def edge(e):
            x = xb[xbase + e, :]
            evec = jnp.full((16,), xbase + e, jnp.int32)
            for k in range(9):
                bk = plsc.load_gather(
                    cb, [evec, jnp.full((16,), k, jnp.int32)])
                ub[ubase + e, pl.ds(k * 16, 16)] = x * bk

    def fire_scatter(xq, uh):
        pltpu.async_copy(ub.at[pl.ds(uh * WB, WB), :],
                         acc.at[rbx.at[xq]], sem_u, add=True)

    def drain_scatter(xq, uh):
        pltpu.make_async_copy(ub.at[pl.ds(uh * WB, WB), :],
                              acc.at[rbx.at[xq]], sem_u).wait()

    def qpass(q, carry):
        cglob = ci * CPS + q

        for j in range(ROWS_PT // ZROWS):
            pltpu.async_copy(
                zbuf, acc.at[pl.ds(si * ROWS_PT + j * ZROWS, ZROWS), :],
                sem_e)
        for j in range(ROWS_PT // ZROWS):
            pltpu.make_async_copy(
                zbuf, acc.at[pl.ds(si * ROWS_PT + j * ZROWS, ZROWS), :],
                sem_e).wait()
        plsc.subcore_barrier()

        def win4(t, carry2):
            w0 = t * 4
            for h in range(4):
                fire_erec(w0 + h, h)
            for h in range(4):
                pltpu.make_async_copy(
                    coef.at[pl.ds(0, WB), :], cb.at[pl.ds(h * WB, WB), :],
                    sem_e).wait()
            build_and_fire_gather(0, cglob, sem_x)
            build_and_fire_gather(1, cglob, sem_x)
            build_and_fire_gather(2, cglob, sem_y)
            build_and_fire_gather(3, cglob, sem_y)
            for h in range(2):
                pltpu.make_async_copy(
                    nodes16.at[gix.at[h]], xb.at[pl.ds(h * WB, WB), :],
                    sem_x).wait()
            compute(0, 0)
            fire_scatter(0, 0)
            compute(1, 1)
            fire_scatter(1, 1)
            for h in range(2, 4):
                pltpu.make_async_copy(
                    nodes16.at[gix.at[h]], xb.at[pl.ds(h * WB, WB), :],
                    sem_y).wait()
            drain_scatter(0, 0)
            drain_scatter(1, 1)
            compute(2, 0)
            fire_scatter(2, 0)
            compute(3, 1)
            fire_scatter(3, 1)
            drain_scatter(2, 0)
            drain_scatter(3, 1)
            return carry2

        lax.fori_loop(0, NWIN // 4, win4, 0)

        fire_erec(NWIN - 1, 0)
        pltpu.make_async_copy(coef.at[pl.ds(0, WB), :],
                              cb.at[pl.ds(0, WB), :], sem_e).wait()
        build_and_fire_gather(0, cglob, sem_x)
        pltpu.make_async_copy(nodes16.at[gix.at[0]],
                              xb.at[pl.ds(0, WB), :], sem_x).wait()
        compute(0, 0)
        pltpu.sync_copy(ub.at[pl.ds(0, WB), :], acc.at[rbx.at[0]], add=True)

        plsc.subcore_barrier()

        for k in range(9):
            pltpu.async_copy(
                acc.at[pl.ds(si * ROWS_PT, ROWS_PT), pl.ds(k * 16, 16)],
                agg_out.at[k, pl.ds(si * ROWS_PT, ROWS_PT),
                           pl.ds(cglob * 16, 16)], sem_e)
        for k in range(9):
            pltpu.make_async_copy(
                acc.at[pl.ds(si * ROWS_PT, ROWS_PT), pl.ds(k * 16, 16)],
                agg_out.at[k, pl.ds(si * ROWS_PT, ROWS_PT),
                           pl.ds(cglob * 16, 16)], sem_e).wait()
        plsc.subcore_barrier()
        return carry

    lax.fori_loop(0, CPS, qpass, 0)


def _prep_body(wpre1, wpost1, wpre2, wpost2, b1_ref, b2_ref):
    w1 = jnp.dot(wpre1[...], wpost1[...],
                 preferred_element_type=jnp.float32) * (INV * INV)
    w2 = jnp.dot(wpre2[...], wpost2[...],
                 preferred_element_type=jnp.float32) * (INV * INV)
    cols1 = lax.broadcasted_iota(jnp.int32, (D, 3 * D), 1)
    rows1 = lax.broadcasted_iota(jnp.int32, (D, 3 * D), 0)
    b1_parts = []
    for i in range(3):
        p = (cols1 == 3 * rows1 + i).astype(jnp.float32)
        b1_parts.append(jnp.dot(w1, p, preferred_element_type=jnp.float32))
    b1_ref[...] = jnp.concatenate(b1_parts, axis=0)
    cols2 = lax.broadcasted_iota(jnp.int32, (D, 5 * D), 1)
    rows2 = lax.broadcasted_iota(jnp.int32, (D, 5 * D), 0)
    b2_parts = []
    for i in range(5):
        p = (cols2 == 5 * rows2 + i).astype(jnp.float32)
        b2_parts.append(jnp.dot(w2, p, preferred_element_type=jnp.float32))
    b2_ref[...] = jnp.concatenate(b2_parts, axis=0)


def _main_body(agg, nodes_blk, wpre0, wpost0, wsc, b1, b2, out_ref):
    a = agg[...]
    s_agg = a[0] * (1.0 / DEN)
    h = jax.nn.gelu(jnp.dot(s_agg, wpre0[...],
                            preferred_element_type=jnp.float32) * INV)
    s_out = jnp.dot(h, wpost0[...], preferred_element_type=jnp.float32) * INV
    s_out = s_out + jnp.dot(nodes_blk[...], wsc[...],
                            preferred_element_type=jnp.float32) * INV
    cat1 = jnp.concatenate([a[1], a[2], a[3]], axis=1) * (1.0 / DEN)
    v1 = jnp.dot(cat1, b1[...], preferred_element_type=jnp.float32)
    cat2 = jnp.concatenate([a[4], a[5], a[6], a[7], a[8]], axis=1) * (1.0 / DEN)
    v2 = jnp.dot(cat2, b2[...], preferred_element_type=jnp.float32)
    out_ref[...] = jnp.concatenate([s_out, v1, v2], axis=1)


def kernel(nodes, positions, senders, receivers,
           W_pre0, W_pre1, W_pre2, W_post0, W_post1, W_post2, W_sc):
    senders = senders.astype(jnp.int32)
    receivers = receivers.astype(jnp.int32)
    posx = positions[:, 0]
    posy = positions[:, 1]
    posz = positions[:, 2]
    nodes16 = nodes.reshape(N * 8, 16)

    mesh = plsc.VectorSubcoreMesh(core_axis_name="c", subcore_axis_name="s")

    sc_params = pltpu.CompilerParams(needs_layout_passes=False,
                                     use_tc_tiling_on_sc=False)

    coef = pl.kernel(
        _coef_body,
        mesh=mesh,
        compiler_params=sc_params,
        out_type=jax.ShapeDtypeStruct((E, 16), jnp.float32),
        scratch_types=[
            pltpu.VMEM((N,), jnp.float32),
            pltpu.VMEM((N,), jnp.float32),
            pltpu.VMEM((N,), jnp.float32),
            pltpu.VMEM((WA,), jnp.int32),
            pltpu.VMEM((WA,), jnp.int32),
            pltpu.VMEM((WA, 16), jnp.float32),
        ],
    )(posx, posy, posz, senders, receivers)

    agg = pl.kernel(
        _agg_body,
        mesh=mesh,
        compiler_params=sc_params,
        out_type=jax.ShapeDtypeStruct((9, N, D), jnp.float32),
        scratch_types=[
            pltpu.VMEM_SHARED((N, 144), jnp.float32),
            pltpu.VMEM((4 * WB, 16), jnp.float32),
            pltpu.VMEM((4 * WB, 16), jnp.float32),
            pltpu.VMEM((2 * WB, 144), jnp.float32),
            pltpu.VMEM((4, WB), jnp.int32),
            pltpu.VMEM((4, WB), jnp.int32),
            pltpu.VMEM((ZROWS, 144), jnp.float32),
            pltpu.SemaphoreType.DMA,
            pltpu.SemaphoreType.DMA,
            pltpu.SemaphoreType.DMA,
            pltpu.SemaphoreType.DMA,
        ],
    )(nodes16, coef)

    b1, b2 = pl.pallas_call(
        _prep_body,
        out_shape=(jax.ShapeDtypeStruct((3 * D, 3 * D), jnp.float32),
                   jax.ShapeDtypeStruct((5 * D, 5 * D), jnp.float32)),
    )(W_pre1, W_post1, W_pre2, W_post2)

    BN = 1000
    out = pl.pallas_call(
        _main_body,
        grid=(N // BN,),
        in_specs=[
            pl.BlockSpec((9, BN, D), lambda i: (0, i, 0)),
            pl.BlockSpec((BN, D), lambda i: (i, 0)),
            pl.BlockSpec((D, D), lambda i: (0, 0)),
            pl.BlockSpec((D, D), lambda i: (0, 0)),
            pl.BlockSpec((D, D), lambda i: (0, 0)),
            pl.BlockSpec((3 * D, 3 * D), lambda i: (0, 0)),
            pl.BlockSpec((5 * D, 5 * D), lambda i: (0, 0)),
        ],
        out_specs=pl.BlockSpec((BN, 9 * D), lambda i: (i, 0)),
        out_shape=jax.ShapeDtypeStruct((N, 9 * D), jnp.float32),
    )(agg, nodes, W_pre0, W_post0, W_sc, b1, b2)
    return out


# speedup vs baseline: 6.3115x; 1.0168x over previous
import functools

import jax
import jax.numpy as jnp
import numpy as np
from jax import lax
from jax.experimental import pallas as pl
from jax.experimental.pallas import tpu as pltpu
from jax.experimental.pallas import tpu_sc as plsc

N = 10000
E = 160000
D = 128
DEN = 16.0
INV = 1.0 / np.sqrt(D)

NC = 2
NS = 16
NWORK = NC * NS

EPW = E // NWORK
WA = 1000
EPT = E // NS
WB = 80
SUB = 80
NSUB = WB // SUB
NWIN = EPT // WB
ROWS_PT = N // NS
ZROWS = 25
CPS = 4

_S3 = float(np.sqrt(3.0))
_S15 = float(np.sqrt(15.0))
_C22 = float(np.sqrt(5.0) / 2.0)
_C24 = float(np.sqrt(15.0) / 2.0)


def _rsqrt(r2):
    i = plsc.bitcast(r2, jnp.int32)
    i = 0x5F3759DF - lax.shift_right_logical(i, 1)
    y = plsc.bitcast(i, jnp.float32)
    for _ in range(3):
        y = y * (1.5 - 0.5 * r2 * y * y)
    return y


def _coef_body(posx, posy, posz, senders, receivers, coef_out,
               px, py, pz, sidx, ridx, cobuf):
    ci = lax.axis_index("c")
    si = lax.axis_index("s")
    wid = si * NC + ci
    base = wid * EPW
    pltpu.sync_copy(posx, px)
    pltpu.sync_copy(posy, py)
    pltpu.sync_copy(posz, pz)
    lane = lax.iota(jnp.int32, 16)

    def win(w, carry):
        wb = base + w * WA
        pltpu.sync_copy(senders.at[pl.ds(wb, WA)], sidx)
        pltpu.sync_copy(receivers.at[pl.ds(wb, WA)], ridx)

        def grp(g, carry2):
            off = jnp.minimum(g * 16, WA - 16)
            sv = sidx[pl.ds(off, 16)]
            rv = ridx[pl.ds(off, 16)]
            sx = plsc.load_gather(px, [sv])
            sy = plsc.load_gather(py, [sv])
            sz = plsc.load_gather(pz, [sv])
            rx = plsc.load_gather(px, [rv])
            ry = plsc.load_gather(py, [rv])
            rz = plsc.load_gather(pz, [rv])
            dx = rx - sx
            dy = ry - sy
            dz = rz - sz
            r2 = dx * dx + dy * dy + dz * dz + 1e-12
            y = _rsqrt(r2)
            ux = dx * y
            uy = dy * y
            uz = dz * y
            cs = [
                jnp.full((16,), 1.0, jnp.float32),
                _S3 * ux, _S3 * uy, _S3 * uz,
                _S15 * ux * uy, _S15 * uy * uz,
                _C22 * (3.0 * uz * uz - 1.0),
                _S15 * ux * uz, _C24 * (ux * ux - uy * uy),
            ]
            eidx = off + lane
            for k in range(9):
                kvec = jnp.full((16,), k, jnp.int32)
                plsc.store_scatter(cobuf, [eidx, kvec], cs[k])
            plsc.store_scatter(cobuf, [eidx, jnp.full((16,), 9, jnp.int32)],
                               plsc.bitcast(sv * 8, jnp.float32))
            plsc.store_scatter(cobuf, [eidx, jnp.full((16,), 10, jnp.int32)],
                               plsc.bitcast(rv, jnp.float32))
            return carry2

        lax.fori_loop(0, (WA + 15) // 16, grp, 0)
        pltpu.sync_copy(cobuf, coef_out.at[pl.ds(wb, WA), :])
        return carry

    lax.fori_loop(0, EPW // WA, win, 0)


def _agg_body(nodes16, coef, agg_out,
              acc, cb, xb, ub, gix, rbx, zbuf, sem_e, sem_x, sem_u, sem_y):
    ci = lax.axis_index("c")
    si = lax.axis_index("s")
    lane = lax.iota(jnp.int32, 16)

    def zrow(i, carry):
        for k in range(9):
            zbuf[i, pl.ds(k * 16, 16)] = jnp.zeros((16,), jnp.float32)
        return carry

    lax.fori_loop(0, ZROWS, zrow, 0)

    def fire_erec(w, half):
        eb = si * EPT + w * WB
        pltpu.async_copy(coef.at[pl.ds(eb, WB), :],
                         cb.at[pl.ds(half * WB, WB), :], sem_e)

    def build_and_fire_gather(half, cglob, sem):
        hb = half * WB
        for g in range(WB // 16):
            rows = hb + g * 16 + lane
            sv8 = plsc.bitcast(
                plsc.load_gather(cb, [rows, jnp.full((16,), 9, jnp.int32)]),
                jnp.int32)
            gix[half, pl.ds(g * 16, 16)] = sv8 + cglob
            rv = plsc.bitcast(
                plsc.load_gather(cb, [rows, jnp.full((16,), 10, jnp.int32)]),
                jnp.int32)
            rbx[half, pl.ds(g * 16, 16)] = rv
        pltpu.async_copy(nodes16.at[gix.at[half]],
                         xb.at[pl.ds(hb, WB), :], sem)

    def compute(xq, uh):
        xbase = xq * WB
        ubase = uh * WB

        @plsc.parallel_loop(0, WB, unroll=8)
        def edge(e):
            x = xb[xbase + e, :]
            evec = jnp.full((16,), xbase + e, jnp.int32)
            for k in range(9):
                bk = plsc.load_gather(
                    cb, [evec, jnp.full((16,), k, jnp.int32)])
                ub[ubase + e, pl.ds(k * 16, 16)] = x * bk

    def fire_scatter(xq, uh):
        pltpu.async_copy(ub.at[pl.ds(uh * WB, WB), :],
                         acc.at[rbx.at[xq]], sem_u, add=True)

    def drain_scatter(xq, uh):
        pltpu.make_async_copy(ub.at[pl.ds(uh * WB, WB), :],
                              acc.at[rbx.at[xq]], sem_u).wait()

    def qpass(q, carry):
        cglob = ci * CPS + q

        for j in range(ROWS_PT // ZROWS):
            pltpu.async_copy(
                zbuf, acc.at[pl.ds(si * ROWS_PT + j * ZROWS, ZROWS), :],
                sem_e)
        for j in range(ROWS_PT // ZROWS):
            pltpu.make_async_copy(
                zbuf, acc.at[pl.ds(si * ROWS_PT + j * ZROWS, ZROWS), :],
                sem_e).wait()
        plsc.subcore_barrier()

        def win4(t, carry2):
            w0 = t * 4
            fire_erec(w0 + 0, 0)
            fire_erec(w0 + 1, 1)
            pltpu.async_copy(coef.at[pl.ds(si * EPT + (w0 + 2) * WB, WB), :],
                             cb.at[pl.ds(2 * WB, WB), :], sem_u)
            pltpu.async_copy(coef.at[pl.ds(si * EPT + (w0 + 3) * WB, WB), :],
                             cb.at[pl.ds(3 * WB, WB), :], sem_u)
            for h in range(2):
                pltpu.make_async_copy(
                    coef.at[pl.ds(0, WB), :], cb.at[pl.ds(h * WB, WB), :],
                    sem_e).wait()
            build_and_fire_gather(0, cglob, sem_x)
            build_and_fire_gather(1, cglob, sem_x)
            for h in range(2, 4):
                pltpu.make_async_copy(
                    coef.at[pl.ds(0, WB), :], cb.at[pl.ds(h * WB, WB), :],
                    sem_u).wait()
            build_and_fire_gather(2, cglob, sem_y)
            build_and_fire_gather(3, cglob, sem_y)
            for h in range(2):
                pltpu.make_async_copy(
                    nodes16.at[gix.at[h]], xb.at[pl.ds(h * WB, WB), :],
                    sem_x).wait()
            compute(0, 0)
            fire_scatter(0, 0)
            compute(1, 1)
            fire_scatter(1, 1)
            for h in range(2, 4):
                pltpu.make_async_copy(
                    nodes16.at[gix.at[h]], xb.at[pl.ds(h * WB, WB), :],
                    sem_y).wait()
            drain_scatter(0, 0)
            drain_scatter(1, 1)
            compute(2, 0)
            fire_scatter(2, 0)
            compute(3, 1)
            fire_scatter(3, 1)
            drain_scatter(2, 0)
            drain_scatter(3, 1)
            return carry2

        lax.fori_loop(0, NWIN // 4, win4, 0)

        fire_erec(NWIN - 1, 0)
        pltpu.make_async_copy(coef.at[pl.ds(0, WB), :],
                              cb.at[pl.ds(0, WB), :], sem_e).wait()
        build_and_fire_gather(0, cglob, sem_x)
        pltpu.make_async_copy(nodes16.at[gix.at[0]],
                              xb.at[pl.ds(0, WB), :], sem_x).wait()
        compute(0, 0)
        pltpu.sync_copy(ub.at[pl.ds(0, WB), :], acc.at[rbx.at[0]], add=True)

        plsc.subcore_barrier()

        for k in range(9):
            pltpu.async_copy(
                acc.at[pl.ds(si * ROWS_PT, ROWS_PT), pl.ds(k * 16, 16)],
                agg_out.at[k, pl.ds(si * ROWS_PT, ROWS_PT),
                           pl.ds(cglob * 16, 16)], sem_e)
        for k in range(9):
            pltpu.make_async_copy(
                acc.at[pl.ds(si * ROWS_PT, ROWS_PT), pl.ds(k * 16, 16)],
                agg_out.at[k, pl.ds(si * ROWS_PT, ROWS_PT),
                           pl.ds(cglob * 16, 16)], sem_e).wait()
        plsc.subcore_barrier()
        return carry

    lax.fori_loop(0, CPS, qpass, 0)


def _prep_body(wpre1, wpost1, wpre2, wpost2, b1_ref, b2_ref):
    w1 = jnp.dot(wpre1[...], wpost1[...],
                 preferred_element_type=jnp.float32) * (INV * INV)
    w2 = jnp.dot(wpre2[...], wpost2[...],
                 preferred_element_type=jnp.float32) * (INV * INV)
    cols1 = lax.broadcasted_iota(jnp.int32, (D, 3 * D), 1)
    rows1 = lax.broadcasted_iota(jnp.int32, (D, 3 * D), 0)
    b1_parts = []
    for i in range(3):
        p = (cols1 == 3 * rows1 + i).astype(jnp.float32)
        b1_parts.append(jnp.dot(w1, p, preferred_element_type=jnp.float32))
    b1_ref[...] = jnp.concatenate(b1_parts, axis=0)
    cols2 = lax.broadcasted_iota(jnp.int32, (D, 5 * D), 1)
    rows2 = lax.broadcasted_iota(jnp.int32, (D, 5 * D), 0)
    b2_parts = []
    for i in range(5):
        p = (cols2 == 5 * rows2 + i).astype(jnp.float32)
        b2_parts.append(jnp.dot(w2, p, preferred_element_type=jnp.float32))
    b2_ref[...] = jnp.concatenate(b2_parts, axis=0)


def _main_body(agg, nodes_blk, wpre0, wpost0, wsc, b1, b2, out_ref):
    a = agg[...]
    s_agg = a[0] * (1.0 / DEN)
    h = jax.nn.gelu(jnp.dot(s_agg, wpre0[...],
                            preferred_element_type=jnp.float32) * INV)
    s_out = jnp.dot(h, wpost0[...], preferred_element_type=jnp.float32) * INV
    s_out = s_out + jnp.dot(nodes_blk[...], wsc[...],
                            preferred_element_type=jnp.float32) * INV
    cat1 = jnp.concatenate([a[1], a[2], a[3]], axis=1) * (1.0 / DEN)
    v1 = jnp.dot(cat1, b1[...], preferred_element_type=jnp.float32)
    cat2 = jnp.concatenate([a[4], a[5], a[6], a[7], a[8]], axis=1) * (1.0 / DEN)
    v2 = jnp.dot(cat2, b2[...], preferred_element_type=jnp.float32)
    out_ref[...] = jnp.concatenate([s_out, v1, v2], axis=1)


def kernel(nodes, positions, senders, receivers,
           W_pre0, W_pre1, W_pre2, W_post0, W_post1, W_post2, W_sc):
    senders = senders.astype(jnp.int32)
    receivers = receivers.astype(jnp.int32)
    posx = positions[:, 0]
    posy = positions[:, 1]
    posz = positions[:, 2]
    nodes16 = nodes.reshape(N * 8, 16)

    mesh = plsc.VectorSubcoreMesh(core_axis_name="c", subcore_axis_name="s")

    sc_params = pltpu.CompilerParams(needs_layout_passes=False,
                                     use_tc_tiling_on_sc=False)

    coef = pl.kernel(
        _coef_body,
        mesh=mesh,
        compiler_params=sc_params,
        out_type=jax.ShapeDtypeStruct((E, 16), jnp.float32),
        scratch_types=[
            pltpu.VMEM((N,), jnp.float32),
            pltpu.VMEM((N,), jnp.float32),
            pltpu.VMEM((N,), jnp.float32),
            pltpu.VMEM((WA,), jnp.int32),
            pltpu.VMEM((WA,), jnp.int32),
            pltpu.VMEM((WA, 16), jnp.float32),
        ],
    )(posx, posy, posz, senders, receivers)

    agg = pl.kernel(
        _agg_body,
        mesh=mesh,
        compiler_params=sc_params,
        out_type=jax.ShapeDtypeStruct((9, N, D), jnp.float32),
        scratch_types=[
            pltpu.VMEM_SHARED((N, 144), jnp.float32),
            pltpu.VMEM((4 * WB, 16), jnp.float32),
            pltpu.VMEM((4 * WB, 16), jnp.float32),
            pltpu.VMEM((2 * WB, 144), jnp.float32),
            pltpu.VMEM((4, WB), jnp.int32),
            pltpu.VMEM((4, WB), jnp.int32),
            pltpu.VMEM((ZROWS, 144), jnp.float32),
            pltpu.SemaphoreType.DMA,
            pltpu.SemaphoreType.DMA,
            pltpu.SemaphoreType.DMA,
            pltpu.SemaphoreType.DMA,
        ],
    )(nodes16, coef)

    b1, b2 = pl.pallas_call(
        _prep_body,
        out_shape=(jax.ShapeDtypeStruct((3 * D, 3 * D), jnp.float32),
                   jax.ShapeDtypeStruct((5 * D, 5 * D), jnp.float32)),
    )(W_pre1, W_post1, W_pre2, W_post2)

    BN = 1000
    out = pl.pallas_call(
        _main_body,
        grid=(N // BN,),
        in_specs=[
            pl.BlockSpec((9, BN, D), lambda i: (0, i, 0)),
            pl.BlockSpec((BN, D), lambda i: (i, 0)),
            pl.BlockSpec((D, D), lambda i: (0, 0)),
            pl.BlockSpec((D, D), lambda i: (0, 0)),
            pl.BlockSpec((D, D), lambda i: (0, 0)),
            pl.BlockSpec((3 * D, 3 * D), lambda i: (0, 0)),
            pl.BlockSpec((5 * D, 5 * D), lambda i: (0, 0)),
        ],
        out_specs=pl.BlockSpec((BN, 9 * D), lambda i: (i, 0)),
        out_shape=jax.ShapeDtypeStruct((N, 9 * D), jnp.float32),
    )(agg, nodes, W_pre0, W_post0, W_sc, b1, b2)
    return out
